# Initial kernel scaffold; baseline (speedup 1.0000x reference)
import jax
import jax.numpy as jnp
from jax.experimental import pallas as pl


def kernel(x, edge_index, params):
    raise NotImplementedError("write your pallas kernel here")


# speedup vs baseline: 16.2230x; 16.2230x over previous
import functools

import jax
import jax.numpy as jnp
from jax import lax
from jax.experimental import pallas as pl
from jax.experimental.pallas import tpu as pltpu
from jax.experimental.pallas import tpu_sc as plsc

N = 10000
NP = 10240
E = 320000
D = 128
NH = 8
DHD = 16
NC = 2
NS = 16
NW = NC * NS
EPW = E // NW
B = 80
NCH = EPW // B
RPT = NP // NS

_f32 = jnp.float32
_i32 = jnp.int32


def _mesh():
    return plsc.VectorSubcoreMesh(
        core_axis_name="c", subcore_axis_name="s", num_cores=NC, num_subcores=NS
    )


def _zero_vmem(buf, rows, width):
    z = jnp.zeros((16,), _f32)

    @pl.loop(0, rows)
    def _(r):
        for c in range(width // 16):
            buf[r, pl.ds(c * 16, 16)] = z


def _zero_accum(accum, zbuf, zrows, sid):
    base = sid * RPT
    for j in range(RPT // zrows):
        pltpu.sync_copy(zbuf, accum.at[pl.ds(base + j * zrows, zrows)])


def _writeback(accum, out, cid, sid, width):
    row = cid * NP + sid * RPT
    pltpu.sync_copy(accum.at[pl.ds(sid * RPT, RPT)], out.at[pl.ds(row, RPT)])


def _build_sc_deg(interpret=False):
    def body(src, dst, out_in, out_out, ones_v, sidx, didx, acc):
        cid = lax.axis_index("c")
        sid = lax.axis_index("s")

        ebase = (cid * NS + sid) * EPW
        for out_ref, idx_ref in ((out_in, didx), (out_out, sidx)):
            _zero_vmem(ones_v, B, D)
            _zero_accum(acc, ones_v, B, sid)

            @pl.loop(0, B)
            def _(r):
                for c in range(D // 16):
                    ones_v[r, pl.ds(c * 16, 16)] = jnp.ones((16,), _f32)

            plsc.subcore_barrier()

            @pl.loop(0, NCH)
            def _(i):
                off = ebase + i * B
                pltpu.sync_copy(src.at[pl.ds(off, B)], sidx)
                pltpu.sync_copy(dst.at[pl.ds(off, B)], didx)
                pltpu.sync_copy(ones_v, acc.at[idx_ref], add=True)

            plsc.subcore_barrier()
            _writeback(acc, out_ref, cid, sid, D)
            plsc.subcore_barrier()

    return pl.kernel(
        body,
        out_type=[
            jax.ShapeDtypeStruct((NC * NP, D), _f32),
            jax.ShapeDtypeStruct((NC * NP, D), _f32),
        ],
        mesh=_mesh(),
        scratch_types=[
            pltpu.VMEM((B, D), _f32),
            pltpu.VMEM((B,), _i32),
            pltpu.VMEM((B,), _i32),
            pltpu.VMEM_SHARED((NP, D), _f32),
        ],
        interpret=interpret,
    )


def _build_sc_segsum(interpret=False):
    def body(table, src, dst, out, sidx, didx, rows, sem, accum):
        cid = lax.axis_index("c")
        sid = lax.axis_index("s")
        _zero_vmem(rows, B, D)
        _zero_accum(accum, rows, B, sid)
        plsc.subcore_barrier()

        ebase = (cid * NS + sid) * EPW

        @pl.loop(0, NCH)
        def _(i):
            off = ebase + i * B
            pltpu.sync_copy(src.at[pl.ds(off, B)], sidx)
            pltpu.sync_copy(dst.at[pl.ds(off, B)], didx)
            pltpu.async_copy(table.at[sidx], rows, sem).wait()
            pltpu.sync_copy(rows, accum.at[didx], add=True)

        plsc.subcore_barrier()
        _writeback(accum, out, cid, sid, D)

    return pl.kernel(
        body,
        out_type=jax.ShapeDtypeStruct((NC * NP, D), _f32),
        mesh=_mesh(),
        scratch_types=[
            pltpu.VMEM((B,), _i32),
            pltpu.VMEM((B,), _i32),
            pltpu.VMEM((B, D), _f32),
            pltpu.SemaphoreType.DMA,
            pltpu.VMEM_SHARED((NP, D), _f32),
        ],
        interpret=interpret,
    )


def _build_sc_gat1(interpret=False):
    def body(hl, hr, src, dst, t_out, sidx, didx, gl, gr, sem):
        cid = lax.axis_index("c")
        sid = lax.axis_index("s")
        ebase = (cid * NS + sid) * EPW

        @pl.loop(0, NCH)
        def _(i):
            off = ebase + i * B
            pltpu.sync_copy(src.at[pl.ds(off, B)], sidx)
            pltpu.sync_copy(dst.at[pl.ds(off, B)], didx)
            pltpu.async_copy(hl.at[sidx], gl, sem).wait()
            pltpu.async_copy(hr.at[didx], gr, sem).wait()

            @pl.loop(0, B)
            def _(r):
                for c in range(D // 16):
                    t = gl[r, pl.ds(c * 16, 16)] + gr[r, pl.ds(c * 16, 16)]
                    gl[r, pl.ds(c * 16, 16)] = jnp.maximum(t, 0.2 * t)

            pltpu.sync_copy(gl, t_out.at[pl.ds(off, B)])

    return pl.kernel(
        body,
        out_type=jax.ShapeDtypeStruct((E, D), _f32),
        mesh=_mesh(),
        scratch_types=[
            pltpu.VMEM((B,), _i32),
            pltpu.VMEM((B,), _i32),
            pltpu.VMEM((B, D), _f32),
            pltpu.VMEM((B, D), _f32),
            pltpu.SemaphoreType.DMA,
        ],
        interpret=interpret,
    )


def _build_sc_segsum_lin(interpret=False):
    def body(rows_hbm, dst, out, didx, rows, acc):
        cid = lax.axis_index("c")
        sid = lax.axis_index("s")
        _zero_vmem(rows, B, D)
        _zero_accum(acc, rows, B, sid)
        plsc.subcore_barrier()

        ebase = (cid * NS + sid) * EPW

        @pl.loop(0, NCH)
        def _(i):
            off = ebase + i * B
            pltpu.sync_copy(dst.at[pl.ds(off, B)], didx)
            pltpu.sync_copy(rows_hbm.at[pl.ds(off, B)], rows)
            pltpu.sync_copy(rows, acc.at[didx], add=True)

        plsc.subcore_barrier()
        _writeback(acc, out, cid, sid, D)

    return pl.kernel(
        body,
        out_type=jax.ShapeDtypeStruct((NC * NP, D), _f32),
        mesh=_mesh(),
        scratch_types=[
            pltpu.VMEM((B,), _i32),
            pltpu.VMEM((B, D), _f32),
            pltpu.VMEM_SHARED((NP, D), _f32),
        ],
        interpret=interpret,
    )


def _build_tc_gatlog(interpret=False):
    EB = 4000

    def body(t_ref, aw_ref, pm_ref, ex_o):
        logits = jnp.dot(t_ref[...], aw_ref[...], preferred_element_type=_f32)
        ex_o[...] = jnp.dot(
            jnp.exp(logits), pm_ref[...], preferred_element_type=_f32
        )

    return pl.pallas_call(
        body,
        grid=(E // EB,),
        in_specs=[
            pl.BlockSpec((EB, D), lambda i: (i, 0)),
            pl.BlockSpec((D, NH), lambda i: (0, 0)),
            pl.BlockSpec((NH, D), lambda i: (0, 0)),
        ],
        out_specs=pl.BlockSpec((EB, D), lambda i: (i, 0)),
        out_shape=jax.ShapeDtypeStruct((E, D), _f32),
        interpret=interpret,
    )


def _build_sc_gat2(interpret=False):
    def body(hl, src, dst, ex_hbm, den0, den1, out,
             sidx, didx, gl, ex_v, dn0, dn1, sem, accum):
        cid = lax.axis_index("c")
        sid = lax.axis_index("s")
        _zero_vmem(gl, B, D)
        _zero_accum(accum, gl, B, sid)
        plsc.subcore_barrier()

        ebase = (cid * NS + sid) * EPW

        @pl.loop(0, NCH)
        def _(i):
            off = ebase + i * B
            pltpu.sync_copy(src.at[pl.ds(off, B)], sidx)
            pltpu.sync_copy(dst.at[pl.ds(off, B)], didx)
            pltpu.async_copy(hl.at[sidx], gl, sem).wait()
            pltpu.sync_copy(ex_hbm.at[pl.ds(off, B)], ex_v)
            pltpu.async_copy(den0.at[didx], dn0, sem).wait()
            pltpu.async_copy(den1.at[didx], dn1, sem).wait()

            @pl.loop(0, B)
            def _(r):
                for c in range(D // 16):
                    s = pl.ds(c * 16, 16)
                    gl[r, s] = gl[r, s] * ex_v[r, s] / (dn0[r, s] + dn1[r, s])

            pltpu.sync_copy(gl, accum.at[didx], add=True)

        plsc.subcore_barrier()
        _writeback(accum, out, cid, sid, D)

    return pl.kernel(
        body,
        out_type=jax.ShapeDtypeStruct((NC * NP, D), _f32),
        mesh=_mesh(),
        scratch_types=[
            pltpu.VMEM((B,), _i32),
            pltpu.VMEM((B,), _i32),
            pltpu.VMEM((B, D), _f32),
            pltpu.VMEM((B, D), _f32),
            pltpu.VMEM((B, D), _f32),
            pltpu.VMEM((B, D), _f32),
            pltpu.SemaphoreType.DMA,
            pltpu.VMEM_SHARED((NP, D), _f32),
        ],
        interpret=interpret,
    )


def _ln(x, g, b):
    u = jnp.mean(x, axis=-1, keepdims=True)
    s = jnp.mean((x - u) ** 2, axis=-1, keepdims=True)
    return g * ((x - u) * lax.rsqrt(s + 1e-12)) + b


def _elu(x):
    return jnp.where(x > 0, x, jnp.exp(x) - 1.0)


def _deg_cols(din0, din1):
    return din0[:, 0:1] + din1[:, 0:1]


def _build_tc_prep(interpret=False):
    def body(h_ref, dout_ref, wl_ref, wr_ref, xs_o, hl_o, hr_o):
        h = h_ref[...]
        d = _deg_cols(dout_ref[0], dout_ref[1])[:N]
        inv_out = jnp.where(d > 0, lax.rsqrt(d), 0.0)
        xs_o[...] = h * inv_out
        hl_o[...] = jnp.dot(h, wl_ref[...], preferred_element_type=_f32)
        hr_o[...] = jnp.dot(h, wr_ref[...], preferred_element_type=_f32)

    return pl.pallas_call(
        body,
        out_shape=[jax.ShapeDtypeStruct((N, D), _f32)] * 3,
        interpret=interpret,
    )


def _build_tc_branches(interpret=False):
    R = 2000

    def body(h_ref, sage_ref, conv_ref, hga_ref, din_ref,
             w4_ref, b4_ref, lng_ref, lnb_ref, sew_ref, seb_ref,
             feats_o, z_o):
        h = h_ref[...]
        sage = sage_ref[0] + sage_ref[1]
        conv = conv_ref[0] + conv_ref[1]
        hga = hga_ref[0] + hga_ref[1]
        d = _deg_cols(din_ref[0], din_ref[1])
        inv_mean = 1.0 / jnp.maximum(d, 1.0)
        inv_in = jnp.where(d > 0, lax.rsqrt(d), 0.0)

        hs = (
            jnp.dot(h, w4_ref[0], preferred_element_type=_f32)
            + jnp.dot(sage * inv_mean, w4_ref[1], preferred_element_type=_f32)
            + b4_ref[0]
        )
        hs = _elu(_ln(hs, lng_ref[0], lnb_ref[0]))
        hc = jnp.dot(conv * inv_in, w4_ref[2], preferred_element_type=_f32) + b4_ref[1]
        hc = _elu(_ln(hc, lng_ref[1], lnb_ref[1]))
        hg = hga + b4_ref[2]
        hg = _elu(_ln(hg, lng_ref[2], lnb_ref[2]))
        hsh = jnp.dot(h, w4_ref[3], preferred_element_type=_f32) + b4_ref[3]
        hsh = _elu(_ln(hsh, lng_ref[3], lnb_ref[3]))

        feats = [hs, hc, hg, hsh]
        feats_o[...] = jnp.stack(feats, axis=0)
        zs = []
        for k in range(4):
            z = seb_ref[k]
            for bidx in range(4):
                z = z + jnp.dot(
                    feats[bidx],
                    sew_ref[k, bidx * D:(bidx + 1) * D, :],
                    preferred_element_type=_f32,
                )
            zs.append(z)
        z_o[...] = jnp.stack(zs, axis=0)

    grid = (N // R,)
    full = lambda shape: pl.BlockSpec(shape, lambda i: (0,) * len(shape))
    return pl.pallas_call(
        body,
        grid=grid,
        in_specs=[
            pl.BlockSpec((R, D), lambda i: (i, 0)),
            pl.BlockSpec((2, R, D), lambda i: (0, i, 0)),
            pl.BlockSpec((2, R, D), lambda i: (0, i, 0)),
            pl.BlockSpec((2, R, D), lambda i: (0, i, 0)),
            pl.BlockSpec((2, R, D), lambda i: (0, i, 0)),
            full((4, D, D)),
            full((4, D)),
            full((4, D)),
            full((4, D)),
            full((4, 4 * D, D)),
            full((4, D)),
        ],
        out_specs=[
            pl.BlockSpec((4, R, D), lambda i: (0, i, 0)),
            pl.BlockSpec((4, R, D), lambda i: (0, i, 0)),
        ],
        out_shape=[
            jax.ShapeDtypeStruct((4, N, D), _f32),
            jax.ShapeDtypeStruct((4, N, D), _f32),
        ],
        interpret=interpret,
    )


def _build_tc_stats(interpret=False):
    def body(z_ref, o_ref):
        mus, vrs = [], []
        for k in range(4):
            z = z_ref[k]
            mu = jnp.mean(z, axis=0)
            vrs.append(jnp.mean(z * z, axis=0) - mu * mu)
            mus.append(mu)
        o_ref[...] = jnp.concatenate(
            [jnp.stack(mus, axis=0), jnp.stack(vrs, axis=0)], axis=0
        )

    return pl.pallas_call(
        body,
        out_shape=jax.ShapeDtypeStruct((8, D), _f32),
        interpret=interpret,
    )


def _build_tc_fuse(interpret=False):
    R = 2000

    def body(feats_ref, z_ref, st_ref, seg_ref, sebeta_ref, o_ref):
        o = jnp.zeros((R, D), _f32)
        for k in range(4):
            zn = (z_ref[k] - st_ref[k]) * lax.rsqrt(st_ref[4 + k] + 1e-5)
            w = jax.nn.sigmoid(zn * seg_ref[k] + sebeta_ref[k])
            o = o + feats_ref[k] * w
        o_ref[...] = o

    grid = (N // R,)
    full = lambda shape: pl.BlockSpec(shape, lambda i: (0,) * len(shape))
    return pl.pallas_call(
        body,
        grid=grid,
        in_specs=[
            pl.BlockSpec((4, R, D), lambda i: (0, i, 0)),
            pl.BlockSpec((4, R, D), lambda i: (0, i, 0)),
            full((8, D)),
            full((4, D)),
            full((4, D)),
        ],
        out_specs=pl.BlockSpec((R, D), lambda i: (i, 0)),
        out_shape=jax.ShapeDtypeStruct((N, D), _f32),
        interpret=interpret,
    )


def _build_tc_mha(interpret=False):
    R = 2000
    NCLS = 23

    def body(h1_ref, h2_ref, wqkv_ref, bqkv_ref, m_ref, mt_ref, wc_ref, bc_ref,
             o_ref):
        t = [h1_ref[...], h2_ref[...]]
        q = [jnp.dot(x, wqkv_ref[0], preferred_element_type=_f32) + bqkv_ref[0:1]
             for x in t]
        k = [jnp.dot(x, wqkv_ref[1], preferred_element_type=_f32) + bqkv_ref[1:2]
             for x in t]
        v = [jnp.dot(x, wqkv_ref[2], preferred_element_type=_f32) + bqkv_ref[2:3]
             for x in t]
        scale = 1.0 / (DHD ** 0.5)
        msk = m_ref[...]
        mskt = mt_ref[...]

        def hsum(x):
            return jnp.dot(x, msk, preferred_element_type=_f32)

        def hexp(x):
            return jnp.dot(x, mskt, preferred_element_type=_f32)

        ctxsum = jnp.zeros((R, D), _f32)
        for l in range(2):
            s0 = hsum(q[l] * k[0]) * scale
            s1 = hsum(q[l] * k[1]) * scale
            m = jnp.maximum(s0, s1)
            e0 = jnp.exp(s0 - m)
            e1 = jnp.exp(s1 - m)
            den = hexp(e0 + e1)
            ctxsum = ctxsum + (hexp(e0) * v[0] + hexp(e1) * v[1]) / den
        pooled = 0.5 * ctxsum
        o_ref[...] = (
            jnp.dot(pooled, wc_ref[...], preferred_element_type=_f32) + bc_ref[...]
        )

    grid = (N // R,)
    full = lambda shape: pl.BlockSpec(shape, lambda i: (0,) * len(shape))
    return pl.pallas_call(
        body,
        grid=grid,
        in_specs=[
            pl.BlockSpec((R, D), lambda i: (i, 0)),
            pl.BlockSpec((R, D), lambda i: (i, 0)),
            full((3, D, D)),
            full((3, D)),
            full((D, NH)),
            full((NH, D)),
            full((D, NCLS)),
            full((1, NCLS)),
        ],
        out_specs=pl.BlockSpec((R, NCLS), lambda i: (i, 0)),
        out_shape=jax.ShapeDtypeStruct((N, NCLS), _f32),
        interpret=interpret,
    )


@functools.lru_cache(maxsize=None)
def _fns(interpret=False):
    return dict(
        deg=_build_sc_deg(interpret),
        segsum=_build_sc_segsum(interpret),
        segsum_lin=_build_sc_segsum_lin(interpret),
        gat1=_build_sc_gat1(interpret),
        gat2=_build_sc_gat2(interpret),
        prep=_build_tc_prep(interpret),
        gatlog=_build_tc_gatlog(interpret),
        branches=_build_tc_branches(interpret),
        stats=_build_tc_stats(interpret),
        fuse=_build_tc_fuse(interpret),
        mha=_build_tc_mha(interpret),
    )


def _forward_impl(x, edge_index, params, sc_interpret=False, tc_interpret=False):
    sc = _fns(sc_interpret)
    tc = _fns(tc_interpret)
    src = edge_index[0]
    dst = edge_index[1]

    mask = (jnp.arange(D)[:, None] // DHD == jnp.arange(NH)[None, :]).astype(_f32)
    maskt = mask.T

    din_f, dout_f = sc["deg"](src, dst)
    din = din_f.reshape(NC, NP, D)
    dout = dout_f.reshape(NC, NP, D)

    h = x
    layer_outs = []
    for lp in params["layers"]:
        w4 = jnp.stack(
            [lp["sage"]["Wself"], lp["sage"]["Wneigh"], lp["conv"]["W"], lp["short"]["W"]]
        )
        b4 = jnp.stack([lp["sage"]["b"], lp["conv"]["b"], lp["gat"]["b"], lp["short"]["b"]])
        lng = jnp.stack([lp[k]["ln"]["g"] for k in ("sage", "conv", "gat", "short")])
        lnb = jnp.stack([lp[k]["ln"]["b"] for k in ("sage", "conv", "gat", "short")])
        sew = jnp.stack([se["W"] for se in lp["se"]])
        seb = jnp.stack([se["b"] for se in lp["se"]])
        seg = jnp.stack([se["g"] for se in lp["se"]])
        sebeta = jnp.stack([se["beta"] for se in lp["se"]])

        wl = lp["gat"]["Wl"]
        wr = lp["gat"]["Wr"]
        xs, hl, hr = tc["prep"](h, dout, wl, wr)

        sage_f = sc["segsum"](h, src, dst)
        conv_f = sc["segsum"](xs, src, dst)
        t_rows = sc["gat1"](hl, hr, src, dst)
        aw = mask * lp["gat"]["a"].reshape(D)[:, None]
        ex = tc["gatlog"](t_rows, aw, maskt)
        den_f = sc["segsum_lin"](ex, dst)
        den = den_f.reshape(NC, NP, D)
        hg_f = sc["gat2"](hl, src, dst, ex, den[0], den[1])

        feats, z = tc["branches"](
            h,
            sage_f.reshape(NC, NP, D),
            conv_f.reshape(NC, NP, D),
            hg_f.reshape(NC, NP, D),
            din,
            w4, b4, lng, lnb, sew, seb,
        )
        st = tc["stats"](z)
        h = tc["fuse"](feats, z, st, seg, sebeta)
        layer_outs.append(h)

    mp = params["mha"]
    wqkv = jnp.stack([mp["Wq"], mp["Wk"], mp["Wv"]])
    bqkv = jnp.stack([mp["bq"], mp["bk"], mp["bv"]])
    return tc["mha"](
        layer_outs[0],
        layer_outs[1],
        wqkv,
        bqkv,
        mask,
        maskt,
        params["cls"]["W"],
        params["cls"]["b"].reshape(1, -1),
    )


@jax.jit
def kernel(x, edge_index, params):
    return _forward_impl(x, edge_index, params)


# speedup vs baseline: 33.9473x; 2.0925x over previous
import functools

import jax
import jax.numpy as jnp
from jax import lax
from jax.experimental import pallas as pl
from jax.experimental.pallas import tpu as pltpu
from jax.experimental.pallas import tpu_sc as plsc

N = 10000
NP = 10240
E = 320000
D = 128
NH = 8
DHD = 16
NC = 2
NS = 16
NW = NC * NS
EPW = E // NW
B = 80
NCH = EPW // B
ECH = E // B
RPT = NP // NS

_f32 = jnp.float32
_i32 = jnp.int32


def _mesh():
    return plsc.VectorSubcoreMesh(
        core_axis_name="c", subcore_axis_name="s", num_cores=NC, num_subcores=NS
    )


def _zero_vmem(buf, rows, width):
    z = jnp.zeros((16,), _f32)

    @pl.loop(0, rows)
    def _(r):
        for c in range(width // 16):
            buf[r, pl.ds(c * 16, 16)] = z


def _zero_accum(accum, zbuf, zrows, sid):
    base = sid * RPT
    for j in range(RPT // zrows):
        pltpu.sync_copy(zbuf, accum.at[pl.ds(base + j * zrows, zrows)])


def _writeback(accum, out, cid, sid, width):
    row = cid * NP + sid * RPT
    pltpu.sync_copy(accum.at[pl.ds(sid * RPT, RPT)], out.at[pl.ds(row, RPT)])


def _build_sc_deg(interpret=False):
    def body(src2, dst2, out_in, out_out, ones_v, idx2, acc):
        cid = lax.axis_index("c")
        sid = lax.axis_index("s")

        wid = cid * NS + sid
        for out_ref, idx_hbm in ((out_in, dst2), (out_out, src2)):
            _zero_vmem(ones_v, B, D)
            _zero_accum(acc, ones_v, B, sid)
            pltpu.sync_copy(idx_hbm.at[wid], idx2)

            @pl.loop(0, B)
            def _(r):
                for c in range(D // 16):
                    ones_v[r, pl.ds(c * 16, 16)] = jnp.ones((16,), _f32)

            plsc.subcore_barrier()

            @pl.loop(0, NCH)
            def _(i):
                pltpu.sync_copy(ones_v, acc.at[idx2.at[i]], add=True)

            plsc.subcore_barrier()
            _writeback(acc, out_ref, cid, sid, D)
            plsc.subcore_barrier()

    return pl.kernel(
        body,
        out_type=[
            jax.ShapeDtypeStruct((NC * NP, D), _f32),
            jax.ShapeDtypeStruct((NC * NP, D), _f32),
        ],
        mesh=_mesh(),
        scratch_types=[
            pltpu.VMEM((B, D), _f32),
            pltpu.VMEM((NCH, B), _i32),
            pltpu.VMEM_SHARED((NP, D), _f32),
        ],
        interpret=interpret,
    )


def _build_sc_segsum(interpret=False):
    def body(table, src, dst2, out, sidx, didx, rows0, rows1, sem0, sem1,
             accum):
        cid = lax.axis_index("c")
        sid = lax.axis_index("s")
        _zero_vmem(rows0, B, D)
        _zero_accum(accum, rows0, B, sid)

        wid = cid * NS + sid
        pltpu.sync_copy(src.at[pl.ds(wid * EPW, EPW)], sidx)
        pltpu.sync_copy(dst2.at[wid], didx)
        plsc.subcore_barrier()

        def gidx(i):
            return sidx.at[pl.ds(i * B, B)]

        pltpu.async_copy(table.at[gidx(0)], rows0, sem0)

        @pl.loop(0, (NCH - 1) // 2)
        def _(k):
            i0 = 2 * k
            pltpu.async_copy(table.at[gidx(i0 + 1)], rows1, sem1)
            pltpu.make_async_copy(table.at[gidx(i0)], rows0, sem0).wait()
            pltpu.sync_copy(rows0, accum.at[didx.at[i0]], add=True)
            pltpu.async_copy(table.at[gidx(i0 + 2)], rows0, sem0)
            pltpu.make_async_copy(table.at[gidx(i0 + 1)], rows1, sem1).wait()
            pltpu.sync_copy(rows1, accum.at[didx.at[i0 + 1]], add=True)

        pltpu.make_async_copy(table.at[gidx(NCH - 1)], rows0, sem0).wait()
        pltpu.sync_copy(rows0, accum.at[didx.at[NCH - 1]], add=True)

        plsc.subcore_barrier()
        _writeback(accum, out, cid, sid, D)

    return pl.kernel(
        body,
        out_type=jax.ShapeDtypeStruct((NC * NP, D), _f32),
        mesh=_mesh(),
        scratch_types=[
            pltpu.VMEM((EPW,), _i32),
            pltpu.VMEM((NCH, B), _i32),
            pltpu.VMEM((B, D), _f32),
            pltpu.VMEM((B, D), _f32),
            pltpu.SemaphoreType.DMA,
            pltpu.SemaphoreType.DMA,
            pltpu.VMEM_SHARED((NP, D), _f32),
        ],
        interpret=interpret,
    )


def _build_sc_gat1(interpret=False):
    def body(hl, hr, src2, dst2, t_out, sidx, didx,
             gl0, gr0, gl1, gr1, sem0, sem1):
        cid = lax.axis_index("c")
        sid = lax.axis_index("s")
        wid = cid * NS + sid
        ebase = wid * EPW
        pltpu.sync_copy(src2.at[wid], sidx)
        pltpu.sync_copy(dst2.at[wid], didx)

        def fire(i, gl, gr, sem):
            pltpu.async_copy(hl.at[sidx.at[i]], gl, sem)
            pltpu.async_copy(hr.at[didx.at[i]], gr, sem)

        def drain(i, gl, gr, sem):
            pltpu.make_async_copy(hl.at[sidx.at[i]], gl, sem).wait()
            pltpu.make_async_copy(hr.at[didx.at[i]], gr, sem).wait()

        def compute_store(i, gl, gr):
            @pl.loop(0, B)
            def _(r):
                for c in range(D // 16):
                    t = gl[r, pl.ds(c * 16, 16)] + gr[r, pl.ds(c * 16, 16)]
                    gl[r, pl.ds(c * 16, 16)] = jnp.maximum(t, 0.2 * t)

            pltpu.sync_copy(gl, t_out.at[pl.ds(ebase + i * B, B)])

        fire(0, gl0, gr0, sem0)

        @pl.loop(0, (NCH - 1) // 2)
        def _(k):
            i0 = 2 * k
            fire(i0 + 1, gl1, gr1, sem1)
            drain(i0, gl0, gr0, sem0)
            compute_store(i0, gl0, gr0)
            fire(i0 + 2, gl0, gr0, sem0)
            drain(i0 + 1, gl1, gr1, sem1)
            compute_store(i0 + 1, gl1, gr1)

        drain(NCH - 1, gl0, gr0, sem0)
        compute_store(NCH - 1, gl0, gr0)

    return pl.kernel(
        body,
        out_type=jax.ShapeDtypeStruct((E, D), _f32),
        mesh=_mesh(),
        scratch_types=[
            pltpu.VMEM((NCH, B), _i32),
            pltpu.VMEM((NCH, B), _i32),
            pltpu.VMEM((B, D), _f32),
            pltpu.VMEM((B, D), _f32),
            pltpu.VMEM((B, D), _f32),
            pltpu.VMEM((B, D), _f32),
            pltpu.SemaphoreType.DMA,
            pltpu.SemaphoreType.DMA,
        ],
        interpret=interpret,
    )


def _build_sc_segsum_lin(interpret=False):
    def body(rows_hbm, dst2, out, didx, rows0, rows1, sem0, sem1, acc):
        cid = lax.axis_index("c")
        sid = lax.axis_index("s")
        _zero_vmem(rows0, B, D)
        _zero_accum(acc, rows0, B, sid)

        wid = cid * NS + sid
        ebase = wid * EPW
        pltpu.sync_copy(dst2.at[wid], didx)
        plsc.subcore_barrier()

        def fire(i, rows, sem):
            pltpu.async_copy(rows_hbm.at[pl.ds(ebase + i * B, B)], rows, sem)

        def drain_scatter(i, rows, sem):
            pltpu.make_async_copy(
                rows_hbm.at[pl.ds(ebase + i * B, B)], rows, sem
            ).wait()
            pltpu.sync_copy(rows, acc.at[didx.at[i]], add=True)

        fire(0, rows0, sem0)

        @pl.loop(0, (NCH - 1) // 2)
        def _(k):
            i0 = 2 * k
            fire(i0 + 1, rows1, sem1)
            drain_scatter(i0, rows0, sem0)
            fire(i0 + 2, rows0, sem0)
            drain_scatter(i0 + 1, rows1, sem1)

        drain_scatter(NCH - 1, rows0, sem0)

        plsc.subcore_barrier()
        _writeback(acc, out, cid, sid, D)

    return pl.kernel(
        body,
        out_type=jax.ShapeDtypeStruct((NC * NP, D), _f32),
        mesh=_mesh(),
        scratch_types=[
            pltpu.VMEM((NCH, B), _i32),
            pltpu.VMEM((B, D), _f32),
            pltpu.VMEM((B, D), _f32),
            pltpu.SemaphoreType.DMA,
            pltpu.SemaphoreType.DMA,
            pltpu.VMEM_SHARED((NP, D), _f32),
        ],
        interpret=interpret,
    )


def _build_tc_gatlog(interpret=False):
    EB = 4000

    def body(t_ref, aw_ref, pm_ref, ex_o):
        logits = jnp.dot(t_ref[...], aw_ref[...], preferred_element_type=_f32)
        ex_o[...] = jnp.dot(
            jnp.exp(logits), pm_ref[...], preferred_element_type=_f32
        )

    return pl.pallas_call(
        body,
        grid=(E // EB,),
        in_specs=[
            pl.BlockSpec((EB, D), lambda i: (i, 0)),
            pl.BlockSpec((D, NH), lambda i: (0, 0)),
            pl.BlockSpec((NH, D), lambda i: (0, 0)),
        ],
        out_specs=pl.BlockSpec((EB, D), lambda i: (i, 0)),
        out_shape=jax.ShapeDtypeStruct((E, D), _f32),
        interpret=interpret,
    )


def _build_sc_gat2(interpret=False):
    def body(hl, src, dst2, ex_hbm, den, out,
             sidx, didx, gl, ex_v, dn, sem, accum):
        cid = lax.axis_index("c")
        sid = lax.axis_index("s")
        _zero_vmem(gl, B, D)
        _zero_accum(accum, gl, B, sid)

        wid = cid * NS + sid
        ebase = wid * EPW
        pltpu.sync_copy(src.at[pl.ds(ebase, EPW)], sidx)
        plsc.subcore_barrier()

        @pl.loop(0, NCH)
        def _(i):
            off = ebase + i * B
            gidx = sidx.at[pl.ds(i * B, B)]
            pltpu.async_copy(hl.at[gidx], gl, sem)
            pltpu.async_copy(ex_hbm.at[pl.ds(off, B)], ex_v, sem)
            pltpu.sync_copy(dst2.at[wid * NCH + i], didx)
            pltpu.async_copy(den.at[didx.at[0]], dn, sem)
            pltpu.make_async_copy(hl.at[gidx], gl, sem).wait()
            pltpu.make_async_copy(ex_hbm.at[pl.ds(off, B)], ex_v, sem).wait()
            pltpu.make_async_copy(den.at[didx.at[0]], dn, sem).wait()

            @pl.loop(0, B)
            def _(r):
                for c in range(D // 16):
                    s = pl.ds(c * 16, 16)
                    gl[r, s] = gl[r, s] * ex_v[r, s] / dn[r, s]

            pltpu.sync_copy(gl, accum.at[didx.at[0]], add=True)

        plsc.subcore_barrier()
        _writeback(accum, out, cid, sid, D)

    return pl.kernel(
        body,
        out_type=jax.ShapeDtypeStruct((NC * NP, D), _f32),
        mesh=_mesh(),
        scratch_types=[
            pltpu.VMEM((EPW,), _i32),
            pltpu.VMEM((1, B), _i32),
            pltpu.VMEM((B, D), _f32),
            pltpu.VMEM((B, D), _f32),
            pltpu.VMEM((B, D), _f32),
            pltpu.SemaphoreType.DMA,
            pltpu.VMEM_SHARED((NP, D), _f32),
        ],
        interpret=interpret,
    )


def _ln(x, g, b):
    u = jnp.mean(x, axis=-1, keepdims=True)
    s = jnp.mean((x - u) ** 2, axis=-1, keepdims=True)
    return g * ((x - u) * lax.rsqrt(s + 1e-12)) + b


def _elu(x):
    return jnp.where(x > 0, x, jnp.exp(x) - 1.0)


def _deg_cols(din0, din1):
    return din0[:, 0:1] + din1[:, 0:1]


def _build_tc_addp(interpret=False):
    def body(f_ref, o_ref):
        o_ref[...] = f_ref[0] + f_ref[1]

    return pl.pallas_call(
        body,
        out_shape=jax.ShapeDtypeStruct((NP, D), _f32),
        interpret=interpret,
    )


def _build_tc_prep(interpret=False):
    def body(h_ref, dout_ref, wl_ref, wr_ref, xs_o, hl_o, hr_o):
        h = h_ref[...]
        d = _deg_cols(dout_ref[0], dout_ref[1])[:N]
        inv_out = jnp.where(d > 0, lax.rsqrt(d), 0.0)
        xs_o[...] = h * inv_out
        hl_o[...] = jnp.dot(h, wl_ref[...], preferred_element_type=_f32)
        hr_o[...] = jnp.dot(h, wr_ref[...], preferred_element_type=_f32)

    return pl.pallas_call(
        body,
        out_shape=[jax.ShapeDtypeStruct((N, D), _f32)] * 3,
        interpret=interpret,
    )


def _build_tc_branches(interpret=False):
    R = 2000

    def body(h_ref, sage_ref, conv_ref, hga_ref, din_ref,
             w4_ref, b4_ref, lng_ref, lnb_ref, sew_ref, seb_ref,
             feats_o, z_o):
        h = h_ref[...]
        sage = sage_ref[0] + sage_ref[1]
        conv = conv_ref[0] + conv_ref[1]
        hga = hga_ref[0] + hga_ref[1]
        d = _deg_cols(din_ref[0], din_ref[1])
        inv_mean = 1.0 / jnp.maximum(d, 1.0)
        inv_in = jnp.where(d > 0, lax.rsqrt(d), 0.0)

        hs = (
            jnp.dot(h, w4_ref[0], preferred_element_type=_f32)
            + jnp.dot(sage * inv_mean, w4_ref[1], preferred_element_type=_f32)
            + b4_ref[0]
        )
        hs = _elu(_ln(hs, lng_ref[0], lnb_ref[0]))
        hc = jnp.dot(conv * inv_in, w4_ref[2], preferred_element_type=_f32) + b4_ref[1]
        hc = _elu(_ln(hc, lng_ref[1], lnb_ref[1]))
        hg = hga + b4_ref[2]
        hg = _elu(_ln(hg, lng_ref[2], lnb_ref[2]))
        hsh = jnp.dot(h, w4_ref[3], preferred_element_type=_f32) + b4_ref[3]
        hsh = _elu(_ln(hsh, lng_ref[3], lnb_ref[3]))

        feats = [hs, hc, hg, hsh]
        feats_o[...] = jnp.stack(feats, axis=0)
        zs = []
        for k in range(4):
            z = seb_ref[k]
            for bidx in range(4):
                z = z + jnp.dot(
                    feats[bidx],
                    sew_ref[k, bidx * D:(bidx + 1) * D, :],
                    preferred_element_type=_f32,
                )
            zs.append(z)
        z_o[...] = jnp.stack(zs, axis=0)

    grid = (N // R,)
    full = lambda shape: pl.BlockSpec(shape, lambda i: (0,) * len(shape))
    return pl.pallas_call(
        body,
        grid=grid,
        in_specs=[
            pl.BlockSpec((R, D), lambda i: (i, 0)),
            pl.BlockSpec((2, R, D), lambda i: (0, i, 0)),
            pl.BlockSpec((2, R, D), lambda i: (0, i, 0)),
            pl.BlockSpec((2, R, D), lambda i: (0, i, 0)),
            pl.BlockSpec((2, R, D), lambda i: (0, i, 0)),
            full((4, D, D)),
            full((4, D)),
            full((4, D)),
            full((4, D)),
            full((4, 4 * D, D)),
            full((4, D)),
        ],
        out_specs=[
            pl.BlockSpec((4, R, D), lambda i: (0, i, 0)),
            pl.BlockSpec((4, R, D), lambda i: (0, i, 0)),
        ],
        out_shape=[
            jax.ShapeDtypeStruct((4, N, D), _f32),
            jax.ShapeDtypeStruct((4, N, D), _f32),
        ],
        interpret=interpret,
    )


def _build_tc_stats(interpret=False):
    def body(z_ref, o_ref):
        mus, vrs = [], []
        for k in range(4):
            z = z_ref[k]
            mu = jnp.mean(z, axis=0)
            vrs.append(jnp.mean(z * z, axis=0) - mu * mu)
            mus.append(mu)
        o_ref[...] = jnp.concatenate(
            [jnp.stack(mus, axis=0), jnp.stack(vrs, axis=0)], axis=0
        )

    return pl.pallas_call(
        body,
        out_shape=jax.ShapeDtypeStruct((8, D), _f32),
        interpret=interpret,
    )


def _build_tc_fuse(interpret=False):
    R = 2000

    def body(feats_ref, z_ref, st_ref, seg_ref, sebeta_ref, o_ref):
        o = jnp.zeros((R, D), _f32)
        for k in range(4):
            zn = (z_ref[k] - st_ref[k]) * lax.rsqrt(st_ref[4 + k] + 1e-5)
            w = jax.nn.sigmoid(zn * seg_ref[k] + sebeta_ref[k])
            o = o + feats_ref[k] * w
        o_ref[...] = o

    grid = (N // R,)
    full = lambda shape: pl.BlockSpec(shape, lambda i: (0,) * len(shape))
    return pl.pallas_call(
        body,
        grid=grid,
        in_specs=[
            pl.BlockSpec((4, R, D), lambda i: (0, i, 0)),
            pl.BlockSpec((4, R, D), lambda i: (0, i, 0)),
            full((8, D)),
            full((4, D)),
            full((4, D)),
        ],
        out_specs=pl.BlockSpec((R, D), lambda i: (i, 0)),
        out_shape=jax.ShapeDtypeStruct((N, D), _f32),
        interpret=interpret,
    )


def _build_tc_mha(interpret=False):
    R = 2000
    NCLS = 23

    def body(h1_ref, h2_ref, wqkv_ref, bqkv_ref, m_ref, mt_ref, wc_ref, bc_ref,
             o_ref):
        t = [h1_ref[...], h2_ref[...]]
        q = [jnp.dot(x, wqkv_ref[0], preferred_element_type=_f32) + bqkv_ref[0:1]
             for x in t]
        k = [jnp.dot(x, wqkv_ref[1], preferred_element_type=_f32) + bqkv_ref[1:2]
             for x in t]
        v = [jnp.dot(x, wqkv_ref[2], preferred_element_type=_f32) + bqkv_ref[2:3]
             for x in t]
        scale = 1.0 / (DHD ** 0.5)
        msk = m_ref[...]
        mskt = mt_ref[...]

        def hsum(x):
            return jnp.dot(x, msk, preferred_element_type=_f32)

        def hexp(x):
            return jnp.dot(x, mskt, preferred_element_type=_f32)

        ctxsum = jnp.zeros((R, D), _f32)
        for l in range(2):
            s0 = hsum(q[l] * k[0]) * scale
            s1 = hsum(q[l] * k[1]) * scale
            m = jnp.maximum(s0, s1)
            e0 = jnp.exp(s0 - m)
            e1 = jnp.exp(s1 - m)
            den = hexp(e0 + e1)
            ctxsum = ctxsum + (hexp(e0) * v[0] + hexp(e1) * v[1]) / den
        pooled = 0.5 * ctxsum
        o_ref[...] = (
            jnp.dot(pooled, wc_ref[...], preferred_element_type=_f32) + bc_ref[...]
        )

    grid = (N // R,)
    full = lambda shape: pl.BlockSpec(shape, lambda i: (0,) * len(shape))
    return pl.pallas_call(
        body,
        grid=grid,
        in_specs=[
            pl.BlockSpec((R, D), lambda i: (i, 0)),
            pl.BlockSpec((R, D), lambda i: (i, 0)),
            full((3, D, D)),
            full((3, D)),
            full((D, NH)),
            full((NH, D)),
            full((D, NCLS)),
            full((1, NCLS)),
        ],
        out_specs=pl.BlockSpec((R, NCLS), lambda i: (i, 0)),
        out_shape=jax.ShapeDtypeStruct((N, NCLS), _f32),
        interpret=interpret,
    )


@functools.lru_cache(maxsize=None)
def _fns(interpret=False):
    return dict(
        deg=_build_sc_deg(interpret),
        segsum=_build_sc_segsum(interpret),
        segsum_lin=_build_sc_segsum_lin(interpret),
        gat1=_build_sc_gat1(interpret),
        gat2=_build_sc_gat2(interpret),
        prep=_build_tc_prep(interpret),
        addp=_build_tc_addp(interpret),
        gatlog=_build_tc_gatlog(interpret),
        branches=_build_tc_branches(interpret),
        stats=_build_tc_stats(interpret),
        fuse=_build_tc_fuse(interpret),
        mha=_build_tc_mha(interpret),
    )


def _forward_impl(x, edge_index, params, sc_interpret=False, tc_interpret=False):
    sc = _fns(sc_interpret)
    tc = _fns(tc_interpret)
    src = edge_index[0]
    dst = edge_index[1]
    src_w = src.reshape(NW, NCH, B)
    dst_w = dst.reshape(NW, NCH, B)
    dst_c = dst.reshape(ECH, 1, B)

    mask = (jnp.arange(D)[:, None] // DHD == jnp.arange(NH)[None, :]).astype(_f32)
    maskt = mask.T

    din_f, dout_f = sc["deg"](src_w, dst_w)
    din = din_f.reshape(NC, NP, D)
    dout = dout_f.reshape(NC, NP, D)

    h = x
    layer_outs = []
    for lp in params["layers"]:
        w4 = jnp.stack(
            [lp["sage"]["Wself"], lp["sage"]["Wneigh"], lp["conv"]["W"], lp["short"]["W"]]
        )
        b4 = jnp.stack([lp["sage"]["b"], lp["conv"]["b"], lp["gat"]["b"], lp["short"]["b"]])
        lng = jnp.stack([lp[k]["ln"]["g"] for k in ("sage", "conv", "gat", "short")])
        lnb = jnp.stack([lp[k]["ln"]["b"] for k in ("sage", "conv", "gat", "short")])
        sew = jnp.stack([se["W"] for se in lp["se"]])
        seb = jnp.stack([se["b"] for se in lp["se"]])
        seg = jnp.stack([se["g"] for se in lp["se"]])
        sebeta = jnp.stack([se["beta"] for se in lp["se"]])

        wl = lp["gat"]["Wl"]
        wr = lp["gat"]["Wr"]
        xs, hl, hr = tc["prep"](h, dout, wl, wr)

        sage_f = sc["segsum"](h, src, dst_w)
        conv_f = sc["segsum"](xs, src, dst_w)
        t_rows = sc["gat1"](hl, hr, src_w, dst_w)
        aw = mask * lp["gat"]["a"].reshape(D)[:, None]
        ex = tc["gatlog"](t_rows, aw, maskt)
        den_f = sc["segsum_lin"](ex, dst_w)
        den = tc["addp"](den_f.reshape(NC, NP, D))
        hg_f = sc["gat2"](hl, src, dst_c, ex, den)

        feats, z = tc["branches"](
            h,
            sage_f.reshape(NC, NP, D),
            conv_f.reshape(NC, NP, D),
            hg_f.reshape(NC, NP, D),
            din,
            w4, b4, lng, lnb, sew, seb,
        )
        st = tc["stats"](z)
        h = tc["fuse"](feats, z, st, seg, sebeta)
        layer_outs.append(h)

    mp = params["mha"]
    wqkv = jnp.stack([mp["Wq"], mp["Wk"], mp["Wv"]])
    bqkv = jnp.stack([mp["bq"], mp["bk"], mp["bv"]])
    return tc["mha"](
        layer_outs[0],
        layer_outs[1],
        wqkv,
        bqkv,
        mask,
        maskt,
        params["cls"]["W"],
        params["cls"]["b"].reshape(1, -1),
    )


@jax.jit
def kernel(x, edge_index, params):
    return _forward_impl(x, edge_index, params)


# speedup vs baseline: 39.5351x; 1.1646x over previous
import functools

import jax
import jax.numpy as jnp
from jax import lax
from jax.experimental import pallas as pl
from jax.experimental.pallas import tpu as pltpu
from jax.experimental.pallas import tpu_sc as plsc

N = 10000
NP = 10240
E = 320000
D = 128
NH = 8
DHD = 16
NC = 2
NS = 16
NW = NC * NS
EPW = E // NW
B = 80
NCH = EPW // B
ECH = E // B
RPT = NP // NS

_f32 = jnp.float32
_i32 = jnp.int32


def _mesh():
    return plsc.VectorSubcoreMesh(
        core_axis_name="c", subcore_axis_name="s", num_cores=NC, num_subcores=NS
    )


def _zero_vmem(buf, rows, width):
    z = jnp.zeros((16,), _f32)

    @pl.loop(0, rows)
    def _(r):
        for c in range(width // 16):
            buf[r, pl.ds(c * 16, 16)] = z


def _zero_accum(accum, zbuf, zrows, sid):
    base = sid * RPT
    for j in range(RPT // zrows):
        pltpu.sync_copy(zbuf, accum.at[pl.ds(base + j * zrows, zrows)])


def _writeback(accum, out, cid, sid, width):
    row = cid * NP + sid * RPT
    pltpu.sync_copy(accum.at[pl.ds(sid * RPT, RPT)], out.at[pl.ds(row, RPT)])


def _build_sc_deg(interpret=False):
    def body(src2, dst2, out_in, out_out, ones_v, idx2, acc):
        cid = lax.axis_index("c")
        sid = lax.axis_index("s")

        wid = cid * NS + sid
        for out_ref, idx_hbm in ((out_in, dst2), (out_out, src2)):
            _zero_vmem(ones_v, B, D)
            _zero_accum(acc, ones_v, B, sid)
            pltpu.sync_copy(idx_hbm.at[wid], idx2)

            @pl.loop(0, B)
            def _(r):
                for c in range(D // 16):
                    ones_v[r, pl.ds(c * 16, 16)] = jnp.ones((16,), _f32)

            plsc.subcore_barrier()

            @pl.loop(0, NCH)
            def _(i):
                pltpu.sync_copy(ones_v, acc.at[idx2.at[i]], add=True)

            plsc.subcore_barrier()
            _writeback(acc, out_ref, cid, sid, D)
            plsc.subcore_barrier()

    return pl.kernel(
        body,
        out_type=[
            jax.ShapeDtypeStruct((NC * NP, D), _f32),
            jax.ShapeDtypeStruct((NC * NP, D), _f32),
        ],
        mesh=_mesh(),
        scratch_types=[
            pltpu.VMEM((B, D), _f32),
            pltpu.VMEM((NCH, B), _i32),
            pltpu.VMEM_SHARED((NP, D), _f32),
        ],
        interpret=interpret,
    )


def _build_sc_segsum(interpret=False):
    def body(table, src, dst2, out, sidx, didx, rows0, rows1, sem0, sem1,
             accum):
        cid = lax.axis_index("c")
        sid = lax.axis_index("s")
        _zero_vmem(rows0, B, D)
        _zero_accum(accum, rows0, B, sid)

        wid = cid * NS + sid
        pltpu.sync_copy(src.at[pl.ds(wid * EPW, EPW)], sidx)
        pltpu.sync_copy(dst2.at[wid], didx)
        plsc.subcore_barrier()

        def gidx(i):
            return sidx.at[pl.ds(i * B, B)]

        pltpu.async_copy(table.at[gidx(0)], rows0, sem0)

        @pl.loop(0, (NCH - 1) // 2)
        def _(k):
            i0 = 2 * k
            pltpu.async_copy(table.at[gidx(i0 + 1)], rows1, sem1)
            pltpu.make_async_copy(table.at[gidx(i0)], rows0, sem0).wait()
            pltpu.sync_copy(rows0, accum.at[didx.at[i0]], add=True)
            pltpu.async_copy(table.at[gidx(i0 + 2)], rows0, sem0)
            pltpu.make_async_copy(table.at[gidx(i0 + 1)], rows1, sem1).wait()
            pltpu.sync_copy(rows1, accum.at[didx.at[i0 + 1]], add=True)

        pltpu.make_async_copy(table.at[gidx(NCH - 1)], rows0, sem0).wait()
        pltpu.sync_copy(rows0, accum.at[didx.at[NCH - 1]], add=True)

        plsc.subcore_barrier()
        _writeback(accum, out, cid, sid, D)

    return pl.kernel(
        body,
        out_type=jax.ShapeDtypeStruct((NC * NP, D), _f32),
        mesh=_mesh(),
        scratch_types=[
            pltpu.VMEM((EPW,), _i32),
            pltpu.VMEM((NCH, B), _i32),
            pltpu.VMEM((B, D), _f32),
            pltpu.VMEM((B, D), _f32),
            pltpu.SemaphoreType.DMA,
            pltpu.SemaphoreType.DMA,
            pltpu.VMEM_SHARED((NP, D), _f32),
        ],
        interpret=interpret,
    )


def _build_sc_gat1(interpret=False):
    def body(hl, hr, src2, dst2, t_out, sidx, didx,
             gl0, gr0, gl1, gr1, sem0, sem1):
        cid = lax.axis_index("c")
        sid = lax.axis_index("s")
        wid = cid * NS + sid
        ebase = wid * EPW
        pltpu.sync_copy(src2.at[wid], sidx)
        pltpu.sync_copy(dst2.at[wid], didx)

        def fire(i, gl, gr, sem):
            pltpu.async_copy(hl.at[sidx.at[i]], gl, sem)
            pltpu.async_copy(hr.at[didx.at[i]], gr, sem)

        def drain(i, gl, gr, sem):
            pltpu.make_async_copy(hl.at[sidx.at[i]], gl, sem).wait()
            pltpu.make_async_copy(hr.at[didx.at[i]], gr, sem).wait()

        def compute_store(i, gl, gr):
            @pl.loop(0, B)
            def _(r):
                for c in range(D // 16):
                    t = gl[r, pl.ds(c * 16, 16)] + gr[r, pl.ds(c * 16, 16)]
                    gl[r, pl.ds(c * 16, 16)] = jnp.maximum(t, 0.2 * t)

            pltpu.sync_copy(gl, t_out.at[pl.ds(ebase + i * B, B)])

        fire(0, gl0, gr0, sem0)

        @pl.loop(0, (NCH - 1) // 2)
        def _(k):
            i0 = 2 * k
            fire(i0 + 1, gl1, gr1, sem1)
            drain(i0, gl0, gr0, sem0)
            compute_store(i0, gl0, gr0)
            fire(i0 + 2, gl0, gr0, sem0)
            drain(i0 + 1, gl1, gr1, sem1)
            compute_store(i0 + 1, gl1, gr1)

        drain(NCH - 1, gl0, gr0, sem0)
        compute_store(NCH - 1, gl0, gr0)

    return pl.kernel(
        body,
        out_type=jax.ShapeDtypeStruct((E, D), _f32),
        mesh=_mesh(),
        scratch_types=[
            pltpu.VMEM((NCH, B), _i32),
            pltpu.VMEM((NCH, B), _i32),
            pltpu.VMEM((B, D), _f32),
            pltpu.VMEM((B, D), _f32),
            pltpu.VMEM((B, D), _f32),
            pltpu.VMEM((B, D), _f32),
            pltpu.SemaphoreType.DMA,
            pltpu.SemaphoreType.DMA,
        ],
        interpret=interpret,
    )


def _build_sc_segsum_lin(interpret=False):
    def body(rows_hbm, dst2, out, didx, rows0, rows1, sem0, sem1, acc):
        cid = lax.axis_index("c")
        sid = lax.axis_index("s")
        _zero_vmem(rows0, B, D)
        _zero_accum(acc, rows0, B, sid)

        wid = cid * NS + sid
        ebase = wid * EPW
        pltpu.sync_copy(dst2.at[wid], didx)
        plsc.subcore_barrier()

        def fire(i, rows, sem):
            pltpu.async_copy(rows_hbm.at[pl.ds(ebase + i * B, B)], rows, sem)

        def drain_scatter(i, rows, sem):
            pltpu.make_async_copy(
                rows_hbm.at[pl.ds(ebase + i * B, B)], rows, sem
            ).wait()
            pltpu.sync_copy(rows, acc.at[didx.at[i]], add=True)

        fire(0, rows0, sem0)

        @pl.loop(0, (NCH - 1) // 2)
        def _(k):
            i0 = 2 * k
            fire(i0 + 1, rows1, sem1)
            drain_scatter(i0, rows0, sem0)
            fire(i0 + 2, rows0, sem0)
            drain_scatter(i0 + 1, rows1, sem1)

        drain_scatter(NCH - 1, rows0, sem0)

        plsc.subcore_barrier()
        _writeback(acc, out, cid, sid, D)

    return pl.kernel(
        body,
        out_type=jax.ShapeDtypeStruct((NC * NP, D), _f32),
        mesh=_mesh(),
        scratch_types=[
            pltpu.VMEM((NCH, B), _i32),
            pltpu.VMEM((B, D), _f32),
            pltpu.VMEM((B, D), _f32),
            pltpu.SemaphoreType.DMA,
            pltpu.SemaphoreType.DMA,
            pltpu.VMEM_SHARED((NP, D), _f32),
        ],
        interpret=interpret,
    )


def _build_tc_gatlog(interpret=False):
    EB = 4000

    def body(t_ref, aw_ref, pm_ref, ex_o):
        logits = jnp.dot(t_ref[...], aw_ref[...], preferred_element_type=_f32)
        ex_o[...] = jnp.dot(
            jnp.exp(logits), pm_ref[...], preferred_element_type=_f32
        )

    return pl.pallas_call(
        body,
        grid=(E // EB,),
        in_specs=[
            pl.BlockSpec((EB, D), lambda i: (i, 0)),
            pl.BlockSpec((D, NH), lambda i: (0, 0)),
            pl.BlockSpec((NH, D), lambda i: (0, 0)),
        ],
        out_specs=pl.BlockSpec((EB, D), lambda i: (i, 0)),
        out_shape=jax.ShapeDtypeStruct((E, D), _f32),
        interpret=interpret,
    )


def _build_sc_gat2(interpret=False):
    def body(hl, src, dst2, ex_hbm, out,
             sidx, didx, gl0, gl1, ex_v, sem0, sem1, semx, accum):
        cid = lax.axis_index("c")
        sid = lax.axis_index("s")
        _zero_vmem(gl0, B, D)
        _zero_accum(accum, gl0, B, sid)

        wid = cid * NS + sid
        ebase = wid * EPW
        pltpu.sync_copy(src.at[pl.ds(ebase, EPW)], sidx)
        plsc.subcore_barrier()

        def fire(i, gl, sem):
            pltpu.async_copy(hl.at[sidx.at[pl.ds(i * B, B)]], gl, sem)

        def work(i, gl, sem):
            off = ebase + i * B
            pltpu.async_copy(ex_hbm.at[pl.ds(off, B)], ex_v, semx)
            pltpu.sync_copy(dst2.at[wid * NCH + i], didx)
            pltpu.make_async_copy(hl.at[sidx.at[pl.ds(i * B, B)]], gl, sem).wait()
            pltpu.make_async_copy(ex_hbm.at[pl.ds(off, B)], ex_v, semx).wait()

            @pl.loop(0, B)
            def _(r):
                for c in range(D // 16):
                    s = pl.ds(c * 16, 16)
                    gl[r, s] = gl[r, s] * ex_v[r, s]

            pltpu.sync_copy(gl, accum.at[didx.at[0]], add=True)

        fire(0, gl0, sem0)

        @pl.loop(0, (NCH - 1) // 2)
        def _(k):
            i0 = 2 * k
            fire(i0 + 1, gl1, sem1)
            work(i0, gl0, sem0)
            fire(i0 + 2, gl0, sem0)
            work(i0 + 1, gl1, sem1)

        work(NCH - 1, gl0, sem0)

        plsc.subcore_barrier()
        _writeback(accum, out, cid, sid, D)

    return pl.kernel(
        body,
        out_type=jax.ShapeDtypeStruct((NC * NP, D), _f32),
        mesh=_mesh(),
        scratch_types=[
            pltpu.VMEM((EPW,), _i32),
            pltpu.VMEM((1, B), _i32),
            pltpu.VMEM((B, D), _f32),
            pltpu.VMEM((B, D), _f32),
            pltpu.VMEM((B, D), _f32),
            pltpu.SemaphoreType.DMA,
            pltpu.SemaphoreType.DMA,
            pltpu.SemaphoreType.DMA,
            pltpu.VMEM_SHARED((NP, D), _f32),
        ],
        interpret=interpret,
    )


def _ln(x, g, b):
    u = jnp.mean(x, axis=-1, keepdims=True)
    s = jnp.mean((x - u) ** 2, axis=-1, keepdims=True)
    return g * ((x - u) * lax.rsqrt(s + 1e-12)) + b


def _elu(x):
    return jnp.where(x > 0, x, jnp.exp(x) - 1.0)


def _deg_cols(din0, din1):
    return din0[:, 0:1] + din1[:, 0:1]


def _build_tc_addp(interpret=False):
    def body(f_ref, o_ref):
        o_ref[...] = f_ref[0] + f_ref[1]

    return pl.pallas_call(
        body,
        out_shape=jax.ShapeDtypeStruct((NP, D), _f32),
        interpret=interpret,
    )


def _build_tc_prep(interpret=False):
    def body(h_ref, dout_ref, wl_ref, wr_ref, xs_o, hl_o, hr_o):
        h = h_ref[...]
        d = _deg_cols(dout_ref[0], dout_ref[1])[:N]
        inv_out = jnp.where(d > 0, lax.rsqrt(d), 0.0)
        xs_o[...] = h * inv_out
        hl_o[...] = jnp.dot(h, wl_ref[...], preferred_element_type=_f32)
        hr_o[...] = jnp.dot(h, wr_ref[...], preferred_element_type=_f32)

    return pl.pallas_call(
        body,
        out_shape=[jax.ShapeDtypeStruct((N, D), _f32)] * 3,
        interpret=interpret,
    )


def _build_tc_branches(interpret=False):
    R = 2000

    def body(h_ref, sage_ref, conv_ref, hga_ref, den_ref, din_ref,
             w4_ref, b4_ref, lng_ref, lnb_ref, sew_ref, seb_ref,
             feats_o, z_o):
        h = h_ref[...]
        sage = sage_ref[0] + sage_ref[1]
        conv = conv_ref[0] + conv_ref[1]
        den = den_ref[0] + den_ref[1]
        hga = jnp.where(den > 0, (hga_ref[0] + hga_ref[1]) / den, 0.0)
        d = _deg_cols(din_ref[0], din_ref[1])
        inv_mean = 1.0 / jnp.maximum(d, 1.0)
        inv_in = jnp.where(d > 0, lax.rsqrt(d), 0.0)

        hs = (
            jnp.dot(h, w4_ref[0], preferred_element_type=_f32)
            + jnp.dot(sage * inv_mean, w4_ref[1], preferred_element_type=_f32)
            + b4_ref[0]
        )
        hs = _elu(_ln(hs, lng_ref[0], lnb_ref[0]))
        hc = jnp.dot(conv * inv_in, w4_ref[2], preferred_element_type=_f32) + b4_ref[1]
        hc = _elu(_ln(hc, lng_ref[1], lnb_ref[1]))
        hg = hga + b4_ref[2]
        hg = _elu(_ln(hg, lng_ref[2], lnb_ref[2]))
        hsh = jnp.dot(h, w4_ref[3], preferred_element_type=_f32) + b4_ref[3]
        hsh = _elu(_ln(hsh, lng_ref[3], lnb_ref[3]))

        feats = [hs, hc, hg, hsh]
        feats_o[...] = jnp.stack(feats, axis=0)
        zs = []
        for k in range(4):
            z = seb_ref[k]
            for bidx in range(4):
                z = z + jnp.dot(
                    feats[bidx],
                    sew_ref[k, bidx * D:(bidx + 1) * D, :],
                    preferred_element_type=_f32,
                )
            zs.append(z)
        z_o[...] = jnp.stack(zs, axis=0)

    grid = (N // R,)
    full = lambda shape: pl.BlockSpec(shape, lambda i: (0,) * len(shape))
    return pl.pallas_call(
        body,
        grid=grid,
        in_specs=[
            pl.BlockSpec((R, D), lambda i: (i, 0)),
            pl.BlockSpec((2, R, D), lambda i: (0, i, 0)),
            pl.BlockSpec((2, R, D), lambda i: (0, i, 0)),
            pl.BlockSpec((2, R, D), lambda i: (0, i, 0)),
            pl.BlockSpec((2, R, D), lambda i: (0, i, 0)),
            pl.BlockSpec((2, R, D), lambda i: (0, i, 0)),
            full((4, D, D)),
            full((4, D)),
            full((4, D)),
            full((4, D)),
            full((4, 4 * D, D)),
            full((4, D)),
        ],
        out_specs=[
            pl.BlockSpec((4, R, D), lambda i: (0, i, 0)),
            pl.BlockSpec((4, R, D), lambda i: (0, i, 0)),
        ],
        out_shape=[
            jax.ShapeDtypeStruct((4, N, D), _f32),
            jax.ShapeDtypeStruct((4, N, D), _f32),
        ],
        interpret=interpret,
    )


def _build_tc_stats(interpret=False):
    def body(z_ref, o_ref):
        mus, vrs = [], []
        for k in range(4):
            z = z_ref[k]
            mu = jnp.mean(z, axis=0)
            vrs.append(jnp.mean(z * z, axis=0) - mu * mu)
            mus.append(mu)
        o_ref[...] = jnp.concatenate(
            [jnp.stack(mus, axis=0), jnp.stack(vrs, axis=0)], axis=0
        )

    return pl.pallas_call(
        body,
        out_shape=jax.ShapeDtypeStruct((8, D), _f32),
        interpret=interpret,
    )


def _build_tc_fuse(interpret=False):
    R = 2000

    def body(feats_ref, z_ref, st_ref, seg_ref, sebeta_ref, o_ref):
        o = jnp.zeros((R, D), _f32)
        for k in range(4):
            zn = (z_ref[k] - st_ref[k]) * lax.rsqrt(st_ref[4 + k] + 1e-5)
            w = jax.nn.sigmoid(zn * seg_ref[k] + sebeta_ref[k])
            o = o + feats_ref[k] * w
        o_ref[...] = o

    grid = (N // R,)
    full = lambda shape: pl.BlockSpec(shape, lambda i: (0,) * len(shape))
    return pl.pallas_call(
        body,
        grid=grid,
        in_specs=[
            pl.BlockSpec((4, R, D), lambda i: (0, i, 0)),
            pl.BlockSpec((4, R, D), lambda i: (0, i, 0)),
            full((8, D)),
            full((4, D)),
            full((4, D)),
        ],
        out_specs=pl.BlockSpec((R, D), lambda i: (i, 0)),
        out_shape=jax.ShapeDtypeStruct((N, D), _f32),
        interpret=interpret,
    )


def _build_tc_mha(interpret=False):
    R = 2000
    NCLS = 23

    def body(h1_ref, h2_ref, wqkv_ref, bqkv_ref, m_ref, mt_ref, wc_ref, bc_ref,
             o_ref):
        t = [h1_ref[...], h2_ref[...]]
        q = [jnp.dot(x, wqkv_ref[0], preferred_element_type=_f32) + bqkv_ref[0:1]
             for x in t]
        k = [jnp.dot(x, wqkv_ref[1], preferred_element_type=_f32) + bqkv_ref[1:2]
             for x in t]
        v = [jnp.dot(x, wqkv_ref[2], preferred_element_type=_f32) + bqkv_ref[2:3]
             for x in t]
        scale = 1.0 / (DHD ** 0.5)
        msk = m_ref[...]
        mskt = mt_ref[...]

        def hsum(x):
            return jnp.dot(x, msk, preferred_element_type=_f32)

        def hexp(x):
            return jnp.dot(x, mskt, preferred_element_type=_f32)

        ctxsum = jnp.zeros((R, D), _f32)
        for l in range(2):
            s0 = hsum(q[l] * k[0]) * scale
            s1 = hsum(q[l] * k[1]) * scale
            m = jnp.maximum(s0, s1)
            e0 = jnp.exp(s0 - m)
            e1 = jnp.exp(s1 - m)
            den = hexp(e0 + e1)
            ctxsum = ctxsum + (hexp(e0) * v[0] + hexp(e1) * v[1]) / den
        pooled = 0.5 * ctxsum
        o_ref[...] = (
            jnp.dot(pooled, wc_ref[...], preferred_element_type=_f32) + bc_ref[...]
        )

    grid = (N // R,)
    full = lambda shape: pl.BlockSpec(shape, lambda i: (0,) * len(shape))
    return pl.pallas_call(
        body,
        grid=grid,
        in_specs=[
            pl.BlockSpec((R, D), lambda i: (i, 0)),
            pl.BlockSpec((R, D), lambda i: (i, 0)),
            full((3, D, D)),
            full((3, D)),
            full((D, NH)),
            full((NH, D)),
            full((D, NCLS)),
            full((1, NCLS)),
        ],
        out_specs=pl.BlockSpec((R, NCLS), lambda i: (i, 0)),
        out_shape=jax.ShapeDtypeStruct((N, NCLS), _f32),
        interpret=interpret,
    )


@functools.lru_cache(maxsize=None)
def _fns(interpret=False):
    return dict(
        deg=_build_sc_deg(interpret),
        segsum=_build_sc_segsum(interpret),
        segsum_lin=_build_sc_segsum_lin(interpret),
        gat1=_build_sc_gat1(interpret),
        gat2=_build_sc_gat2(interpret),
        prep=_build_tc_prep(interpret),
        addp=_build_tc_addp(interpret),
        gatlog=_build_tc_gatlog(interpret),
        branches=_build_tc_branches(interpret),
        stats=_build_tc_stats(interpret),
        fuse=_build_tc_fuse(interpret),
        mha=_build_tc_mha(interpret),
    )


def _forward_impl(x, edge_index, params, sc_interpret=False, tc_interpret=False):
    sc = _fns(sc_interpret)
    tc = _fns(tc_interpret)
    src = edge_index[0]
    dst = edge_index[1]
    src_w = src.reshape(NW, NCH, B)
    dst_w = dst.reshape(NW, NCH, B)
    dst_c = dst.reshape(ECH, 1, B)

    mask = (jnp.arange(D)[:, None] // DHD == jnp.arange(NH)[None, :]).astype(_f32)
    maskt = mask.T

    din_f, dout_f = sc["deg"](src_w, dst_w)
    din = din_f.reshape(NC, NP, D)
    dout = dout_f.reshape(NC, NP, D)

    h = x
    layer_outs = []
    for lp in params["layers"]:
        w4 = jnp.stack(
            [lp["sage"]["Wself"], lp["sage"]["Wneigh"], lp["conv"]["W"], lp["short"]["W"]]
        )
        b4 = jnp.stack([lp["sage"]["b"], lp["conv"]["b"], lp["gat"]["b"], lp["short"]["b"]])
        lng = jnp.stack([lp[k]["ln"]["g"] for k in ("sage", "conv", "gat", "short")])
        lnb = jnp.stack([lp[k]["ln"]["b"] for k in ("sage", "conv", "gat", "short")])
        sew = jnp.stack([se["W"] for se in lp["se"]])
        seb = jnp.stack([se["b"] for se in lp["se"]])
        seg = jnp.stack([se["g"] for se in lp["se"]])
        sebeta = jnp.stack([se["beta"] for se in lp["se"]])

        wl = lp["gat"]["Wl"]
        wr = lp["gat"]["Wr"]
        xs, hl, hr = tc["prep"](h, dout, wl, wr)

        sage_f = sc["segsum"](h, src, dst_w)
        conv_f = sc["segsum"](xs, src, dst_w)
        t_rows = sc["gat1"](hl, hr, src_w, dst_w)
        aw = mask * lp["gat"]["a"].reshape(D)[:, None]
        ex = tc["gatlog"](t_rows, aw, maskt)
        den_f = sc["segsum_lin"](ex, dst_w)
        hg_f = sc["gat2"](hl, src, dst_c, ex)

        feats, z = tc["branches"](
            h,
            sage_f.reshape(NC, NP, D),
            conv_f.reshape(NC, NP, D),
            hg_f.reshape(NC, NP, D),
            den_f.reshape(NC, NP, D),
            din,
            w4, b4, lng, lnb, sew, seb,
        )
        st = tc["stats"](z)
        h = tc["fuse"](feats, z, st, seg, sebeta)
        layer_outs.append(h)

    mp = params["mha"]
    wqkv = jnp.stack([mp["Wq"], mp["Wk"], mp["Wv"]])
    bqkv = jnp.stack([mp["bq"], mp["bk"], mp["bv"]])
    return tc["mha"](
        layer_outs[0],
        layer_outs[1],
        wqkv,
        bqkv,
        mask,
        maskt,
        params["cls"]["W"],
        params["cls"]["b"].reshape(1, -1),
    )


@jax.jit
def kernel(x, edge_index, params):
    return _forward_impl(x, edge_index, params)
